# baseline (speedup 1.0000x reference)
import jax
import jax.numpy as jnp
from jax import lax
from jax.experimental import pallas as pl
from jax.experimental.pallas import tpu as pltpu
from jax.experimental.pallas import tpu_sc as plsc

EMBED_DIM = 128
NUM_CORES = 2
NUM_SUBCORES = 16
NUM_WORKERS = NUM_CORES * NUM_SUBCORES
ROWS_PER_GATHER = 8
NUM_CHUNKS = 2


def _sc_gather_chunk(table, idx, chunk, chunk_batch, seq):
    rows_per_worker = chunk_batch // NUM_WORKERS
    idx_per_worker = rows_per_worker * seq
    gw = ROWS_PER_GATHER * seq
    n_gathers = rows_per_worker // ROWS_PER_GATHER
    chunk_off = chunk * chunk_batch * seq

    mesh = plsc.VectorSubcoreMesh(core_axis_name="core", subcore_axis_name="subcore")

    @pl.kernel(
        out_type=jax.ShapeDtypeStruct((chunk_batch * seq, EMBED_DIM), table.dtype),
        mesh=mesh,
        scratch_types=[
            pltpu.VMEM((idx_per_worker,), jnp.int32),
            pltpu.VMEM((gw, EMBED_DIM), jnp.float32),
            pltpu.VMEM((gw, EMBED_DIM), jnp.float32),
            pltpu.SemaphoreType.DMA,
            pltpu.SemaphoreType.DMA,
            pltpu.SemaphoreType.DMA,
        ],
    )
    def gather_kernel(table_hbm, idx_hbm, out_hbm, idx_v, buf0, buf1, gsem, osem0, osem1):
        wid = lax.axis_index("subcore") * NUM_CORES + lax.axis_index("core")
        row_base = wid * rows_per_worker

        pltpu.sync_copy(
            idx_hbm.at[0, pl.ds(chunk_off + row_base * seq, idx_per_worker)], idx_v
        )

        bufs = (buf0, buf1)
        osems = (osem0, osem1)

        def start_gather(g):
            return pltpu.async_copy(
                table_hbm.at[idx_v.at[pl.ds(g * gw, gw)]], bufs[g % 2], gsem
            )

        out_handles = [None, None]
        gather_handle = start_gather(0)
        for g in range(n_gathers):
            gather_handle.wait()
            if g + 1 < n_gathers:
                nxt = (g + 1) % 2
                if out_handles[nxt] is not None:
                    out_handles[nxt].wait()
                    out_handles[nxt] = None
                gather_handle = start_gather(g + 1)
            out_handles[g % 2] = pltpu.async_copy(
                bufs[g % 2],
                out_hbm.at[pl.ds((row_base + g * ROWS_PER_GATHER) * seq, gw)],
                osems[g % 2],
            )
        for h in out_handles:
            if h is not None:
                h.wait()

    return gather_kernel(table, idx)


def kernel(x, table):
    batch, seq = x.shape
    idx = x.reshape(1, batch * seq).astype(jnp.int32)
    chunk_batch = batch // NUM_CHUNKS

    flats = [
        _sc_gather_chunk(table, idx, c, chunk_batch, seq) for c in range(NUM_CHUNKS)
    ]
    parts = [f.reshape(chunk_batch, seq, EMBED_DIM) for f in flats]
    return jnp.concatenate(parts, axis=0) if len(parts) > 1 else parts[0]


# speedup vs baseline: 1.3864x; 1.3864x over previous
import jax
import jax.numpy as jnp
from jax import lax
from jax.experimental import pallas as pl
from jax.experimental.pallas import tpu as pltpu
from jax.experimental.pallas import tpu_sc as plsc

EMBED_DIM = 128
NUM_CORES = 2
NUM_SUBCORES = 16
NUM_WORKERS = NUM_CORES * NUM_SUBCORES
ROWS_PER_GATHER = 8
NUM_CHUNKS = 1
TC_BLOCK_ROWS = 64


def _sc_gather_chunk(table, idx, chunk, chunk_batch, seq):
    rows_per_worker = chunk_batch // NUM_WORKERS
    idx_per_worker = rows_per_worker * seq
    gw = ROWS_PER_GATHER * seq
    n_gathers = rows_per_worker // ROWS_PER_GATHER
    chunk_off = chunk * chunk_batch * seq

    mesh = plsc.VectorSubcoreMesh(core_axis_name="core", subcore_axis_name="subcore")

    @pl.kernel(
        out_type=jax.ShapeDtypeStruct((chunk_batch * seq, EMBED_DIM), table.dtype),
        mesh=mesh,
        scratch_types=[
            pltpu.VMEM((idx_per_worker,), jnp.int32),
            pltpu.VMEM((gw, EMBED_DIM), jnp.float32),
            pltpu.VMEM((gw, EMBED_DIM), jnp.float32),
            pltpu.SemaphoreType.DMA,
            pltpu.SemaphoreType.DMA,
            pltpu.SemaphoreType.DMA,
        ],
    )
    def gather_kernel(table_hbm, idx_hbm, out_hbm, idx_v, buf0, buf1, gsem, osem0, osem1):
        wid = lax.axis_index("subcore") * NUM_CORES + lax.axis_index("core")
        row_base = wid * rows_per_worker

        pltpu.sync_copy(
            idx_hbm.at[0, pl.ds(chunk_off + row_base * seq, idx_per_worker)], idx_v
        )

        bufs = (buf0, buf1)
        osems = (osem0, osem1)

        def start_gather(g):
            return pltpu.async_copy(
                table_hbm.at[idx_v.at[pl.ds(g * gw, gw)]], bufs[g % 2], gsem
            )

        out_handles = [None, None]
        gather_handle = start_gather(0)
        for g in range(n_gathers):
            gather_handle.wait()
            if g + 1 < n_gathers:
                nxt = (g + 1) % 2
                if out_handles[nxt] is not None:
                    out_handles[nxt].wait()
                    out_handles[nxt] = None
                gather_handle = start_gather(g + 1)
            out_handles[g % 2] = pltpu.async_copy(
                bufs[g % 2],
                out_hbm.at[pl.ds((row_base + g * ROWS_PER_GATHER) * seq, gw)],
                osems[g % 2],
            )
        for h in out_handles:
            if h is not None:
                h.wait()

    return gather_kernel(table, idx)


def _tc_fold_chunk(flat_c, carry, chunk, batch, chunk_batch, seq):
    n_blocks = chunk_batch // TC_BLOCK_ROWS
    in_specs = [pl.BlockSpec((TC_BLOCK_ROWS * seq, EMBED_DIM), lambda i: (i, 0))]
    args = [flat_c]
    io_alias = {}
    if carry is not None:
        in_specs.append(pl.BlockSpec(memory_space=pltpu.MemorySpace.HBM))
        args.append(carry)
        io_alias = {1: 0}

    def body(in_ref, *rest):
        out_ref = rest[-1]
        out_ref[...] = in_ref[...].reshape(TC_BLOCK_ROWS, seq, EMBED_DIM)

    return pl.pallas_call(
        body,
        grid=(n_blocks,),
        in_specs=in_specs,
        out_specs=pl.BlockSpec(
            (TC_BLOCK_ROWS, seq, EMBED_DIM),
            lambda i, c=chunk: (c * n_blocks + i, 0, 0),
        ),
        out_shape=jax.ShapeDtypeStruct((batch, seq, EMBED_DIM), flat_c.dtype),
        input_output_aliases=io_alias,
    )(*args)


def kernel(x, table):
    batch, seq = x.shape
    idx = x.reshape(1, batch * seq).astype(jnp.int32)
    chunk_batch = batch // NUM_CHUNKS

    flats = [
        _sc_gather_chunk(table, idx, c, chunk_batch, seq) for c in range(NUM_CHUNKS)
    ]
    out = None
    for c in range(NUM_CHUNKS):
        out = _tc_fold_chunk(flats[c], out, c, batch, chunk_batch, seq)
    return out
